# baseline (speedup 1.0000x reference)
import functools

import jax
import jax.numpy as jnp
from jax import lax
from jax.experimental import pallas as pl
from jax.experimental.pallas import tpu as pltpu
from jax.experimental.pallas import tpu_sc as plsc

VOCAB = 1000000
EMBED_DIM = 64
PADDED_DIM = 128
BATCH = 16384
HIST_LEN = 20

NC, NS = 2, 16
NW = NC * NS
TB = 128
NTB_W = (BATCH // TB) // NW
BW = TB * NTB_W
NG = EMBED_DIM // 16
GROWS = NG * TB
NBLK = NTB_W * HIST_LEN
NBUF = 2

_mesh = plsc.VectorSubcoreMesh(core_axis_name="c", subcore_axis_name="s")


@functools.partial(
    pl.kernel,
    out_type=jax.ShapeDtypeStruct((HIST_LEN, 8, BATCH // TB, 8, TB),
                                  jnp.float32),
    mesh=_mesh,
    scratch_types=[
        pltpu.VMEM((HIST_LEN, BW), jnp.int32),
        pltpu.VMEM((NBUF, GROWS), jnp.int32),
        pltpu.VMEM((NBUF, GROWS, 16), jnp.float32),
        pltpu.VMEM((NBUF, GROWS, 17), jnp.float32),
        pltpu.VMEM((NBUF, 8, 1, 8, TB), jnp.float32),
        [pltpu.SemaphoreType.DMA] * NBUF,
        [pltpu.SemaphoreType.DMA] * NBUF,
    ],
    compiler_params=pltpu.CompilerParams(use_tc_tiling_on_sc=False,
                                         needs_layout_passes=False),
)
def _gather_kernel(w8_hbm, wordsT_hbm, out_hbm, idx_v, gidx, rows16, rows17,
                   dmaj, gsems, osems):
    wid = lax.axis_index("s") * NC + lax.axis_index("c")
    b0 = wid * BW

    iota = lax.broadcasted_iota(jnp.int32, (16,), 0)

    pltpu.sync_copy(wordsT_hbm.at[:, pl.ds(b0, BW)], idx_v)

    def params(i):
        h = lax.rem(i, HIST_LEN)
        tbl = lax.div(i, HIST_LEN)
        return tbl, h

    def prep_and_gather(i, b):
        tbl, h = params(i)
        gidx_b = gidx.at[b]
        for g in range(NG):
            for k in range(TB // 16):
                v = idx_v[h, pl.ds(tbl * TB + 16 * k, 16)]
                gidx_b[pl.ds(g * TB + 16 * k, 16)] = v * 8 + g
        pltpu.async_copy(w8_hbm.at[gidx_b], rows16.at[b], gsems[b])

    def transpose(b):
        rows_b = rows16.at[b]
        r17_b = rows17.at[b]
        dmaj_b = dmaj.at[b]
        @pl.loop(0, GROWS, unroll=16)
        def _(j):
            r17_b[j, pl.ds(0, 16)] = rows_b[j]

        @pl.loop(0, TB // 16)
        def _(m):
            j16 = iota + 16 * m
            for g in range(NG):
                row16 = j16 + g * TB
                vals = [plsc.load_gather(
                            r17_b, [row16, jnp.full((16,), dl, jnp.int32)])
                        for dl in range(16)]
                for dl in range(16):
                    d = 16 * g + dl
                    dmaj_b[d // 8, 0, d % 8, pl.ds(16 * m, 16)] = vals[dl]

    def store(i, b):
        tbl, h = params(i)
        pltpu.async_copy(
            dmaj.at[b],
            out_hbm.at[h, pl.ds(0, 8), pl.ds(wid * NTB_W + tbl, 1)],
            osems[b])

    def wait_gather(b):
        pltpu.make_async_copy(
            w8_hbm.at[pl.ds(0, GROWS)], rows16.at[b], gsems[b]).wait()

    def wait_store(b):
        pltpu.make_async_copy(
            dmaj.at[b], out_hbm.at[0, pl.ds(0, 8), pl.ds(0, 1)],
            osems[b]).wait()

    for b in range(NBUF):
        prep_and_gather(b, b)
    for b in range(NBUF):
        wait_gather(b)
        transpose(b)
        store(b, b)
        prep_and_gather(b + NBUF, b)

    @pl.loop(0, (NBLK - 2 * NBUF) // NBUF)
    def _(g_):
        for b in range(NBUF):
            i = NBUF + g_ * NBUF + b
            wait_gather(b)
            wait_store(b)
            transpose(b)
            store(i, b)
            prep_and_gather(i + NBUF, b)

    for b in range(NBUF):
        i = NBLK - NBUF + b
        wait_gather(b)
        wait_store(b)
        transpose(b)
        store(i, b)
    for b in range(NBUF):
        wait_store(b)


def kernel(weight, words):
    w8 = jnp.pad(weight, ((0, 0), (0, PADDED_DIM - EMBED_DIM)))
    w8 = w8.reshape(VOCAB * 8, 16)
    wordsT = words.astype(jnp.int32).T
    out5 = _gather_kernel(w8, wordsT)
    return out5.transpose(2, 4, 0, 1, 3).reshape(BATCH, HIST_LEN, EMBED_DIM)


# speedup vs baseline: 1.0468x; 1.0468x over previous
import functools

import jax
import jax.numpy as jnp
from jax import lax
from jax.experimental import pallas as pl
from jax.experimental.pallas import tpu as pltpu
from jax.experimental.pallas import tpu_sc as plsc

VOCAB = 1000000
EMBED_DIM = 64
PADDED_DIM = 128
BATCH = 16384
HIST_LEN = 20

NC, NS = 2, 16
NW = NC * NS
TB = 128
NTB_W = (BATCH // TB) // NW
BW = TB * NTB_W
NG = EMBED_DIM // 16
GROWS = NG * TB
NBLK = NTB_W * HIST_LEN
NBUF = 2

_mesh = plsc.VectorSubcoreMesh(core_axis_name="c", subcore_axis_name="s")


@functools.partial(
    pl.kernel,
    out_type=jax.ShapeDtypeStruct((HIST_LEN, 8, BATCH // TB, 8, TB),
                                  jnp.float32),
    mesh=_mesh,
    scratch_types=[
        pltpu.VMEM((HIST_LEN, BW), jnp.int32),
        pltpu.VMEM((NBUF, GROWS), jnp.int32),
        pltpu.VMEM((NBUF, GROWS, 16), jnp.float32),
        pltpu.VMEM((NBUF, 8, 1, 8, TB), jnp.float32),
        [pltpu.SemaphoreType.DMA] * NBUF,
        [pltpu.SemaphoreType.DMA] * NBUF,
    ],
    compiler_params=pltpu.CompilerParams(use_tc_tiling_on_sc=False,
                                         needs_layout_passes=False),
)
def _gather_kernel(w8_hbm, wordsT_hbm, out_hbm, idx_v, gidx, rows16, dmaj,
                   gsems, osems):
    wid = lax.axis_index("s") * NC + lax.axis_index("c")
    b0 = wid * BW

    iota = lax.broadcasted_iota(jnp.int32, (16,), 0)

    pltpu.sync_copy(wordsT_hbm.at[:, pl.ds(b0, BW)], idx_v)

    def params(i):
        h = lax.rem(i, HIST_LEN)
        tbl = lax.div(i, HIST_LEN)
        return tbl, h

    def prep_and_gather(i, b):
        tbl, h = params(i)
        gidx_b = gidx.at[b]
        for g in range(NG):
            for k in range(TB // 16):
                v = idx_v[h, pl.ds(tbl * TB + 16 * k, 16)]
                gidx_b[pl.ds(g * TB + 16 * k, 16)] = v * 8 + g
        pltpu.async_copy(w8_hbm.at[gidx_b], rows16.at[b], gsems[b])

    def transpose(b):
        rows_b = rows16.at[b]
        dmaj_b = dmaj.at[b]
        for m in range(TB // 16):
            j16 = iota + 16 * m
            for g in range(NG):
                row16 = j16 + g * TB
                vals = [plsc.load_gather(
                            rows_b, [row16, jnp.full((16,), dl, jnp.int32)])
                        for dl in range(16)]
                for dl in range(16):
                    d = 16 * g + dl
                    dmaj_b[d // 8, 0, d % 8, pl.ds(16 * m, 16)] = vals[dl]

    def store(i, b):
        tbl, h = params(i)
        pltpu.async_copy(
            dmaj.at[b],
            out_hbm.at[h, pl.ds(0, 8), pl.ds(wid * NTB_W + tbl, 1)],
            osems[b])

    def wait_gather(b):
        pltpu.make_async_copy(
            w8_hbm.at[pl.ds(0, GROWS)], rows16.at[b], gsems[b]).wait()

    def wait_store(b):
        pltpu.make_async_copy(
            dmaj.at[b], out_hbm.at[0, pl.ds(0, 8), pl.ds(0, 1)],
            osems[b]).wait()

    for b in range(NBUF):
        prep_and_gather(b, b)
    for b in range(NBUF):
        wait_gather(b)
        transpose(b)
        store(b, b)
        prep_and_gather(b + NBUF, b)

    @pl.loop(0, (NBLK - 2 * NBUF) // NBUF)
    def _(g_):
        for b in range(NBUF):
            i = NBUF + g_ * NBUF + b
            wait_gather(b)
            wait_store(b)
            transpose(b)
            store(i, b)
            prep_and_gather(i + NBUF, b)

    for b in range(NBUF):
        i = NBLK - NBUF + b
        wait_gather(b)
        wait_store(b)
        transpose(b)
        store(i, b)
    for b in range(NBUF):
        wait_store(b)


def kernel(weight, words):
    w8 = jnp.pad(weight, ((0, 0), (0, PADDED_DIM - EMBED_DIM)))
    w8 = w8.reshape(VOCAB * 8, 16)
    wordsT = words.astype(jnp.int32).T
    out5 = _gather_kernel(w8, wordsT)
    return out5.transpose(2, 4, 0, 1, 3).reshape(BATCH, HIST_LEN, EMBED_DIM)
